# baseline (speedup 1.0000x reference)
import functools

import jax
import jax.numpy as jnp
from jax import lax
from jax.experimental import pallas as pl
from jax.experimental.pallas import tpu as pltpu
from jax.experimental.pallas import tpu_sc as plsc

N = 10000
E = 160000
D = 256
H = D // 2
NTILES = 16
EDGES_PER_TILE = E // NTILES
CHUNK = 112
NCHUNKS = 90
EDGES_PAD = NCHUNKS * CHUNK
ROWS_PER_TILE = 632
NPAD = ROWS_PER_TILE * NTILES


def _sc_agg_body(xlo, xhi, idx_h, dst_h, zeros_h, dummy_h, lo_out, hi_out,
                 pair_a, pair_b, rows_a, rows_b, dst_v, accum,
                 sem_ia, sem_ib, sem_ga, sem_gb, sem_sa, sem_sb):
    c = lax.axis_index("c")
    s = lax.axis_index("s")

    def fetch_idx(j, pbuf, sem):
        pltpu.async_copy(idx_h.at[s, j], pbuf, sem)

    def gather(pbuf, rbuf, sem):
        @pl.when(c == 0)
        def _():
            pltpu.async_copy(xlo.at[pbuf.at[0]], rbuf, sem)

        @pl.when(c == 1)
        def _():
            pltpu.async_copy(xhi.at[pbuf.at[0]], rbuf, sem)

    def scatter(rbuf, j, sem):
        pltpu.async_copy(rbuf, accum.at[dst_v.at[j]], sem, add=True)

    def wait_idx(pbuf, sem):
        pltpu.make_async_copy(idx_h.at[s, 0], pbuf, sem).wait()

    def wait_rows(rbuf, sem):
        pltpu.make_async_copy(dummy_h, rbuf, sem).wait()

    pltpu.sync_copy(dst_h.at[s], dst_v)
    pltpu.sync_copy(idx_h.at[s, 0], pair_a)
    gather(pair_a, rows_a, sem_ga)
    fetch_idx(1, pair_b, sem_ib)
    pltpu.sync_copy(dummy_h, rows_b)
    pltpu.sync_copy(zeros_h, accum.at[pl.ds(s * ROWS_PER_TILE, ROWS_PER_TILE)])
    plsc.subcore_barrier()
    scatter(rows_b, 0, sem_sb)

    def pair_step(g, carry):
        j = 2 * g
        wait_rows(rows_a, sem_ga)
        scatter(rows_a, j, sem_sa)
        wait_idx(pair_b, sem_ib)
        wait_rows(rows_b, sem_sb)
        gather(pair_b, rows_b, sem_gb)
        fetch_idx((j + 2) % NCHUNKS, pair_a, sem_ia)
        wait_rows(rows_b, sem_gb)
        scatter(rows_b, j + 1, sem_sb)
        wait_idx(pair_a, sem_ia)
        wait_rows(rows_a, sem_sa)
        gather(pair_a, rows_a, sem_ga)
        fetch_idx((j + 3) % NCHUNKS, pair_b, sem_ib)
        return carry

    lax.fori_loop(0, NCHUNKS // 2, pair_step, 0)
    wait_rows(rows_b, sem_sb)
    wait_rows(rows_a, sem_ga)
    wait_idx(pair_b, sem_ib)
    plsc.subcore_barrier()

    row0 = s * ROWS_PER_TILE

    @pl.when(c == 0)
    def _():
        pltpu.sync_copy(accum.at[pl.ds(row0, ROWS_PER_TILE)],
                        lo_out.at[pl.ds(row0, ROWS_PER_TILE)])

    @pl.when(c == 1)
    def _():
        pltpu.sync_copy(accum.at[pl.ds(row0, ROWS_PER_TILE)],
                        hi_out.at[pl.ds(row0, ROWS_PER_TILE)])


_sc_agg = functools.partial(
    pl.kernel,
    mesh=plsc.VectorSubcoreMesh(core_axis_name="c", subcore_axis_name="s"),
    out_type=(jax.ShapeDtypeStruct((NPAD, H), jnp.float32),
              jax.ShapeDtypeStruct((NPAD, H), jnp.float32)),
    scratch_types=[
        pltpu.VMEM((2, CHUNK), jnp.int32),
        pltpu.VMEM((2, CHUNK), jnp.int32),
        pltpu.VMEM((CHUNK, H), jnp.float32),
        pltpu.VMEM((CHUNK, H), jnp.float32),
        pltpu.VMEM((NCHUNKS, CHUNK), jnp.int32),
        pltpu.VMEM_SHARED((NPAD, H), jnp.float32),
        pltpu.SemaphoreType.DMA,
        pltpu.SemaphoreType.DMA,
        pltpu.SemaphoreType.DMA,
        pltpu.SemaphoreType.DMA,
        pltpu.SemaphoreType.DMA,
        pltpu.SemaphoreType.DMA,
    ],
)(_sc_agg_body)


ROWS_BLK = 1000


def _mm_pre1_body(x_r, wl1_r, b1_r, t_r):
    t_r[...] = (jnp.dot(x_r[...], wl1_r[...],
                        preferred_element_type=jnp.float32) + b1_r[...])


def _mm_post1_body(alo_r, ahi_r, t_r, w1_r, lo_r, hi_r):
    h = jnp.dot(alo_r[...], w1_r[:H, :], preferred_element_type=jnp.float32)
    h = h + jnp.dot(ahi_r[...], w1_r[H:, :], preferred_element_type=jnp.float32)
    h = h + t_r[...]
    lo_r[...] = h[:, :H]
    hi_r[...] = h[:, H:]


def _mm_pre2_body(hlo_r, hhi_r, x_r, wl2_r, b2_r, t_r):
    t = jnp.dot(hlo_r[...], wl2_r[:H, :], preferred_element_type=jnp.float32)
    t = t + jnp.dot(hhi_r[...], wl2_r[H:, :], preferred_element_type=jnp.float32)
    t_r[...] = t + b2_r[...] + x_r[...]


def _mm_post2_body(alo_r, ahi_r, t_r, w2_r, out_r):
    h = jnp.dot(alo_r[...], w2_r[:H, :], preferred_element_type=jnp.float32)
    h = h + jnp.dot(ahi_r[...], w2_r[H:, :], preferred_element_type=jnp.float32)
    out_r[...] = (t_r[...] + h) * 0.5


def _row_blk(i):
    return (i, 0)


def _full(i):
    return (0, 0)


_half_spec = pl.BlockSpec((ROWS_BLK, H), _row_blk)
_fullrow_spec = pl.BlockSpec((ROWS_BLK, D), _row_blk)
_w_spec = pl.BlockSpec((D, D), _full)
_b_spec = pl.BlockSpec((1, D), _full)

_mm_pre1 = pl.pallas_call(
    _mm_pre1_body,
    grid=(N // ROWS_BLK,),
    in_specs=[_fullrow_spec, _w_spec, _b_spec],
    out_specs=_fullrow_spec,
    out_shape=jax.ShapeDtypeStruct((N, D), jnp.float32),
)

_mm_post1 = pl.pallas_call(
    _mm_post1_body,
    grid=(N // ROWS_BLK,),
    in_specs=[_half_spec, _half_spec, _fullrow_spec, _w_spec],
    out_specs=[_half_spec, _half_spec],
    out_shape=(jax.ShapeDtypeStruct((N, H), jnp.float32),
               jax.ShapeDtypeStruct((N, H), jnp.float32)),
)

_mm_pre2 = pl.pallas_call(
    _mm_pre2_body,
    grid=(N // ROWS_BLK,),
    in_specs=[_half_spec, _half_spec, _fullrow_spec, _w_spec, _b_spec],
    out_specs=_fullrow_spec,
    out_shape=jax.ShapeDtypeStruct((N, D), jnp.float32),
)

_mm_post2 = pl.pallas_call(
    _mm_post2_body,
    grid=(N // ROWS_BLK,),
    in_specs=[_half_spec, _half_spec, _fullrow_spec, _w_spec],
    out_specs=_fullrow_spec,
    out_shape=jax.ShapeDtypeStruct((N, D), jnp.float32),
)


def kernel(x, edge_index, W1, Wl1, b1, W2, Wl2, b2):
    x_lo = x[:, :H]
    x_hi = x[:, H:]
    npadlen = EDGES_PAD - EDGES_PER_TILE
    pad_src = jnp.broadcast_to(
        jnp.arange(npadlen, dtype=jnp.int32) * 61 % N, (NTILES, npadlen))
    pad_dst = jnp.broadcast_to(
        N + (jnp.arange(npadlen, dtype=jnp.int32) % (NPAD - N)),
        (NTILES, npadlen))
    src_h = jnp.concatenate(
        [edge_index[0].reshape(NTILES, EDGES_PER_TILE), pad_src], axis=1
    ).reshape(NTILES, NCHUNKS, CHUNK)
    dst_h = jnp.concatenate(
        [edge_index[1].reshape(NTILES, EDGES_PER_TILE), pad_dst],
        axis=1,
    ).reshape(NTILES, NCHUNKS, CHUNK)
    idx_h = jnp.stack([src_h, dst_h], axis=2)
    zeros = jnp.zeros((ROWS_PER_TILE, H), jnp.float32)
    dummy = jnp.zeros((CHUNK, H), jnp.float32)
    b1r = b1.reshape(1, D)
    b2r = b2.reshape(1, D)

    t1 = _mm_pre1(x, Wl1, b1r)
    a1lo, a1hi = _sc_agg(x_lo, x_hi, idx_h, dst_h, zeros, dummy)
    h1lo, h1hi = _mm_post1(a1lo[:N], a1hi[:N], t1, W1)
    t2 = _mm_pre2(h1lo, h1hi, x, Wl2, b2r)
    a2lo, a2hi = _sc_agg(h1lo, h1hi, idx_h, dst_h, zeros, dummy)
    return _mm_post2(a2lo[:N], a2hi[:N], t2, W2)


# speedup vs baseline: 1.0519x; 1.0519x over previous
import functools

import jax
import jax.numpy as jnp
from jax import lax
from jax.experimental import pallas as pl
from jax.experimental.pallas import tpu as pltpu
from jax.experimental.pallas import tpu_sc as plsc

N = 10000
E = 160000
D = 256
H = D // 2
NTILES = 16
EDGES_PER_TILE = E // NTILES
CHUNK = 128
NCHUNKS = 80
NPAIRS = NCHUNKS // 2
EDGES_PAD = NCHUNKS * CHUNK
ROWS_PER_TILE = 632
NPAD = ROWS_PER_TILE * NTILES


def _sc_agg_body(xlo, xhi, idx_h, dst_h, zeros_h, dummy_h, lo_out, hi_out,
                 pair_a, pair_b, rows_a, rows_b, dst_v, accum,
                 sem_ia, sem_ib, sem_ga, sem_gb, sem_sa, sem_sb):
    c = lax.axis_index("c")
    s = lax.axis_index("s")

    def fetch_pair(p, pbuf, sem):
        pltpu.async_copy(idx_h.at[s, p], pbuf, sem)

    def gather(idx_row, rbuf, sem):
        @pl.when(c == 0)
        def _():
            pltpu.async_copy(xlo.at[idx_row], rbuf, sem)

        @pl.when(c == 1)
        def _():
            pltpu.async_copy(xhi.at[idx_row], rbuf, sem)

    def scatter(rbuf, j, sem):
        pltpu.async_copy(rbuf, accum.at[dst_v.at[j]], sem, add=True)

    def wait_idx(pbuf, sem):
        pltpu.make_async_copy(idx_h.at[s, 0], pbuf, sem).wait()

    def wait_rows(rbuf, sem):
        pltpu.make_async_copy(dummy_h, rbuf, sem).wait()

    pltpu.sync_copy(dst_h.at[s], dst_v)
    pltpu.sync_copy(idx_h.at[s, 0], pair_a)
    gather(pair_a.at[0], rows_a, sem_ga)
    fetch_pair(1, pair_b, sem_ib)
    pltpu.sync_copy(dummy_h, rows_b)
    pltpu.sync_copy(zeros_h, accum.at[pl.ds(s * ROWS_PER_TILE, ROWS_PER_TILE)])
    plsc.subcore_barrier()
    scatter(rows_b, 0, sem_sb)

    def phase(j, p_cur, p_nxt, sem_i_cur, sem_i_nxt):
        wait_rows(rows_a, sem_ga)
        scatter(rows_a, j, sem_sa)
        wait_rows(rows_b, sem_sb)
        gather(p_cur.at[1], rows_b, sem_gb)
        wait_rows(rows_b, sem_gb)
        scatter(rows_b, j + 1, sem_sb)
        wait_idx(p_nxt, sem_i_nxt)
        wait_rows(rows_a, sem_sa)
        gather(p_nxt.at[0], rows_a, sem_ga)
        fetch_pair(((j + 4) // 2) % NPAIRS, p_cur, sem_i_cur)

    def quad_step(g, carry):
        j = 4 * g
        phase(j, pair_a, pair_b, sem_ia, sem_ib)
        phase(j + 2, pair_b, pair_a, sem_ib, sem_ia)
        return carry

    lax.fori_loop(0, NCHUNKS // 4, quad_step, 0)
    wait_rows(rows_b, sem_sb)
    wait_rows(rows_a, sem_ga)
    wait_idx(pair_b, sem_ib)
    plsc.subcore_barrier()

    row0 = s * ROWS_PER_TILE

    @pl.when(c == 0)
    def _():
        pltpu.sync_copy(accum.at[pl.ds(row0, ROWS_PER_TILE)],
                        lo_out.at[pl.ds(row0, ROWS_PER_TILE)])

    @pl.when(c == 1)
    def _():
        pltpu.sync_copy(accum.at[pl.ds(row0, ROWS_PER_TILE)],
                        hi_out.at[pl.ds(row0, ROWS_PER_TILE)])


_sc_agg = functools.partial(
    pl.kernel,
    mesh=plsc.VectorSubcoreMesh(core_axis_name="c", subcore_axis_name="s"),
    out_type=(jax.ShapeDtypeStruct((NPAD, H), jnp.float32),
              jax.ShapeDtypeStruct((NPAD, H), jnp.float32)),
    scratch_types=[
        pltpu.VMEM((2, CHUNK), jnp.int32),
        pltpu.VMEM((2, CHUNK), jnp.int32),
        pltpu.VMEM((CHUNK, H), jnp.float32),
        pltpu.VMEM((CHUNK, H), jnp.float32),
        pltpu.VMEM((NCHUNKS, CHUNK), jnp.int32),
        pltpu.VMEM_SHARED((NPAD, H), jnp.float32),
        pltpu.SemaphoreType.DMA,
        pltpu.SemaphoreType.DMA,
        pltpu.SemaphoreType.DMA,
        pltpu.SemaphoreType.DMA,
        pltpu.SemaphoreType.DMA,
        pltpu.SemaphoreType.DMA,
    ],
)(_sc_agg_body)


ROWS_BLK = 1000


def _mm_pre1_body(x_r, wl1_r, b1_r, t_r):
    t_r[...] = (jnp.dot(x_r[...], wl1_r[...],
                        preferred_element_type=jnp.float32) + b1_r[...])


def _mm_post1_body(alo_r, ahi_r, t_r, w1_r, lo_r, hi_r):
    h = jnp.dot(alo_r[...], w1_r[:H, :], preferred_element_type=jnp.float32)
    h = h + jnp.dot(ahi_r[...], w1_r[H:, :], preferred_element_type=jnp.float32)
    h = h + t_r[...]
    lo_r[...] = h[:, :H]
    hi_r[...] = h[:, H:]


def _mm_pre2_body(hlo_r, hhi_r, x_r, wl2_r, b2_r, t_r):
    t = jnp.dot(hlo_r[...], wl2_r[:H, :], preferred_element_type=jnp.float32)
    t = t + jnp.dot(hhi_r[...], wl2_r[H:, :], preferred_element_type=jnp.float32)
    t_r[...] = t + b2_r[...] + x_r[...]


def _mm_post2_body(alo_r, ahi_r, t_r, w2_r, out_r):
    h = jnp.dot(alo_r[...], w2_r[:H, :], preferred_element_type=jnp.float32)
    h = h + jnp.dot(ahi_r[...], w2_r[H:, :], preferred_element_type=jnp.float32)
    out_r[...] = (t_r[...] + h) * 0.5


def _row_blk(i):
    return (i, 0)


def _full(i):
    return (0, 0)


_half_spec = pl.BlockSpec((ROWS_BLK, H), _row_blk)
_fullrow_spec = pl.BlockSpec((ROWS_BLK, D), _row_blk)
_w_spec = pl.BlockSpec((D, D), _full)
_b_spec = pl.BlockSpec((1, D), _full)

_mm_pre1 = pl.pallas_call(
    _mm_pre1_body,
    grid=(N // ROWS_BLK,),
    in_specs=[_fullrow_spec, _w_spec, _b_spec],
    out_specs=_fullrow_spec,
    out_shape=jax.ShapeDtypeStruct((N, D), jnp.float32),
)

_mm_post1 = pl.pallas_call(
    _mm_post1_body,
    grid=(N // ROWS_BLK,),
    in_specs=[_half_spec, _half_spec, _fullrow_spec, _w_spec],
    out_specs=[_half_spec, _half_spec],
    out_shape=(jax.ShapeDtypeStruct((N, H), jnp.float32),
               jax.ShapeDtypeStruct((N, H), jnp.float32)),
)

_mm_pre2 = pl.pallas_call(
    _mm_pre2_body,
    grid=(N // ROWS_BLK,),
    in_specs=[_half_spec, _half_spec, _fullrow_spec, _w_spec, _b_spec],
    out_specs=_fullrow_spec,
    out_shape=jax.ShapeDtypeStruct((N, D), jnp.float32),
)

_mm_post2 = pl.pallas_call(
    _mm_post2_body,
    grid=(N // ROWS_BLK,),
    in_specs=[_half_spec, _half_spec, _fullrow_spec, _w_spec],
    out_specs=_fullrow_spec,
    out_shape=jax.ShapeDtypeStruct((N, D), jnp.float32),
)


def kernel(x, edge_index, W1, Wl1, b1, W2, Wl2, b2):
    x_lo = x[:, :H]
    x_hi = x[:, H:]
    npadlen = EDGES_PAD - EDGES_PER_TILE
    pad_src = jnp.broadcast_to(
        jnp.arange(npadlen, dtype=jnp.int32) * 61 % N, (NTILES, npadlen))
    pad_dst = jnp.broadcast_to(
        N + (jnp.arange(npadlen, dtype=jnp.int32) % (NPAD - N)),
        (NTILES, npadlen))
    src_h = jnp.concatenate(
        [edge_index[0].reshape(NTILES, EDGES_PER_TILE), pad_src], axis=1
    ).reshape(NTILES, NCHUNKS, CHUNK)
    dst_h = jnp.concatenate(
        [edge_index[1].reshape(NTILES, EDGES_PER_TILE), pad_dst],
        axis=1,
    ).reshape(NTILES, NCHUNKS, CHUNK)
    idx_h = src_h.reshape(NTILES, NPAIRS, 2, CHUNK)
    zeros = jnp.zeros((ROWS_PER_TILE, H), jnp.float32)
    dummy = jnp.zeros((CHUNK, H), jnp.float32)
    b1r = b1.reshape(1, D)
    b2r = b2.reshape(1, D)

    t1 = _mm_pre1(x, Wl1, b1r)
    a1lo, a1hi = _sc_agg(x_lo, x_hi, idx_h, dst_h, zeros, dummy)
    h1lo, h1hi = _mm_post1(a1lo[:N], a1hi[:N], t1, W1)
    t2 = _mm_pre2(h1lo, h1hi, x, Wl2, b2r)
    a2lo, a2hi = _sc_agg(h1lo, h1hi, idx_h, dst_h, zeros, dummy)
    return _mm_post2(a2lo[:N], a2hi[:N], t2, W2)


# speedup vs baseline: 1.0862x; 1.0326x over previous
import functools

import jax
import jax.numpy as jnp
from jax import lax
from jax.experimental import pallas as pl
from jax.experimental.pallas import tpu as pltpu
from jax.experimental.pallas import tpu_sc as plsc

N = 10000
E = 160000
D = 256
H = D // 2
NTILES = 16
EDGES_PER_TILE = E // NTILES
CHUNK = 128
NCHUNKS = 80
NPAIRS = NCHUNKS // 2
EDGES_PAD = NCHUNKS * CHUNK
ROWS_PER_TILE = 632
NPAD = ROWS_PER_TILE * NTILES


def _sc_agg_body(xlo, xhi, idx_h, dst_h, zeros_h, dummy_h, lo_out, hi_out,
                 pair_a, pair_b, rows_a, rows_b, dst_v, accum,
                 sem_ia, sem_ib, sem_ga, sem_gb, sem_sa, sem_sb):
    c = lax.axis_index("c")
    s = lax.axis_index("s")

    def fetch_pair(p, pbuf, sem):
        pltpu.async_copy(idx_h.at[s, p], pbuf, sem)

    def gather(idx_row, rbuf, sem):
        @pl.when(c == 0)
        def _():
            pltpu.async_copy(xlo.at[idx_row], rbuf, sem)

        @pl.when(c == 1)
        def _():
            pltpu.async_copy(xhi.at[idx_row], rbuf, sem)

    def scatter(rbuf, j, sem):
        pltpu.async_copy(rbuf, accum.at[dst_v.at[j]], sem, add=True)

    def wait_idx(pbuf, sem):
        pltpu.make_async_copy(idx_h.at[s, 0], pbuf, sem).wait()

    def wait_rows(rbuf, sem):
        pltpu.make_async_copy(dummy_h, rbuf, sem).wait()

    pltpu.sync_copy(dst_h.at[s], dst_v)
    pltpu.sync_copy(idx_h.at[s, 0], pair_a)
    gather(pair_a.at[0], rows_a, sem_ga)
    fetch_pair(1, pair_b, sem_ib)
    pltpu.sync_copy(dummy_h, rows_b)
    pltpu.sync_copy(zeros_h, accum.at[pl.ds(s * ROWS_PER_TILE, ROWS_PER_TILE)])
    plsc.subcore_barrier()
    scatter(rows_b, 0, sem_sb)

    def phase(j, p_cur, p_nxt, sem_i_cur, sem_i_nxt):
        wait_rows(rows_a, sem_ga)
        scatter(rows_a, j, sem_sa)
        wait_rows(rows_b, sem_sb)
        gather(p_cur.at[1], rows_b, sem_gb)
        wait_rows(rows_b, sem_gb)
        scatter(rows_b, j + 1, sem_sb)
        wait_idx(p_nxt, sem_i_nxt)
        wait_rows(rows_a, sem_sa)
        gather(p_nxt.at[0], rows_a, sem_ga)
        fetch_pair(((j + 4) // 2) % NPAIRS, p_cur, sem_i_cur)

    def quad_step(g, carry):
        j = 4 * g
        phase(j, pair_a, pair_b, sem_ia, sem_ib)
        phase(j + 2, pair_b, pair_a, sem_ib, sem_ia)
        return carry

    lax.fori_loop(0, NCHUNKS // 4, quad_step, 0)
    wait_rows(rows_b, sem_sb)
    wait_rows(rows_a, sem_ga)
    wait_idx(pair_b, sem_ib)
    plsc.subcore_barrier()

    row0 = s * ROWS_PER_TILE

    @pl.when(c == 0)
    def _():
        pltpu.sync_copy(accum.at[pl.ds(row0, ROWS_PER_TILE)],
                        lo_out.at[pl.ds(row0, ROWS_PER_TILE)])

    @pl.when(c == 1)
    def _():
        pltpu.sync_copy(accum.at[pl.ds(row0, ROWS_PER_TILE)],
                        hi_out.at[pl.ds(row0, ROWS_PER_TILE)])


_sc_agg = functools.partial(
    pl.kernel,
    mesh=plsc.VectorSubcoreMesh(core_axis_name="c", subcore_axis_name="s"),
    out_type=(jax.ShapeDtypeStruct((NPAD, H), jnp.float32),
              jax.ShapeDtypeStruct((NPAD, H), jnp.float32)),
    scratch_types=[
        pltpu.VMEM((2, CHUNK), jnp.int32),
        pltpu.VMEM((2, CHUNK), jnp.int32),
        pltpu.VMEM((CHUNK, H), jnp.float32),
        pltpu.VMEM((CHUNK, H), jnp.float32),
        pltpu.VMEM((NCHUNKS, CHUNK), jnp.int32),
        pltpu.VMEM_SHARED((NPAD, H), jnp.float32),
        pltpu.SemaphoreType.DMA,
        pltpu.SemaphoreType.DMA,
        pltpu.SemaphoreType.DMA,
        pltpu.SemaphoreType.DMA,
        pltpu.SemaphoreType.DMA,
        pltpu.SemaphoreType.DMA,
    ],
)(_sc_agg_body)


ROWS_BLK = 1000


def _mm_pre1_body(x_r, wl1_r, b1_r, t_r):
    t_r[...] = (jnp.dot(x_r[...], wl1_r[...],
                        preferred_element_type=jnp.float32) + b1_r[...])


def _mm_mid_body(alo_r, ahi_r, t_r, w1_r, x_r, wl2_r, b2_r,
                 lo_r, hi_r, t2_r):
    h = jnp.dot(alo_r[...], w1_r[:H, :], preferred_element_type=jnp.float32)
    h = h + jnp.dot(ahi_r[...], w1_r[H:, :], preferred_element_type=jnp.float32)
    h = h + t_r[...]
    lo_r[...] = h[:, :H]
    hi_r[...] = h[:, H:]
    t2 = jnp.dot(h, wl2_r[...], preferred_element_type=jnp.float32)
    t2_r[...] = t2 + b2_r[...] + x_r[...]


def _mm_post2_body(alo_r, ahi_r, t_r, w2_r, out_r):
    h = jnp.dot(alo_r[...], w2_r[:H, :], preferred_element_type=jnp.float32)
    h = h + jnp.dot(ahi_r[...], w2_r[H:, :], preferred_element_type=jnp.float32)
    out_r[...] = (t_r[...] + h) * 0.5


def _row_blk(i):
    return (i, 0)


def _full(i):
    return (0, 0)


_half_spec = pl.BlockSpec((ROWS_BLK, H), _row_blk)
_fullrow_spec = pl.BlockSpec((ROWS_BLK, D), _row_blk)
_w_spec = pl.BlockSpec((D, D), _full)
_b_spec = pl.BlockSpec((1, D), _full)

_mm_pre1 = pl.pallas_call(
    _mm_pre1_body,
    grid=(N // ROWS_BLK,),
    in_specs=[_fullrow_spec, _w_spec, _b_spec],
    out_specs=_fullrow_spec,
    out_shape=jax.ShapeDtypeStruct((N, D), jnp.float32),
)

_mm_mid = pl.pallas_call(
    _mm_mid_body,
    grid=(N // ROWS_BLK,),
    in_specs=[_half_spec, _half_spec, _fullrow_spec, _w_spec, _fullrow_spec,
              _w_spec, _b_spec],
    out_specs=[_half_spec, _half_spec, _fullrow_spec],
    out_shape=(jax.ShapeDtypeStruct((N, H), jnp.float32),
               jax.ShapeDtypeStruct((N, H), jnp.float32),
               jax.ShapeDtypeStruct((N, D), jnp.float32)),
)

_mm_post2 = pl.pallas_call(
    _mm_post2_body,
    grid=(N // ROWS_BLK,),
    in_specs=[_half_spec, _half_spec, _fullrow_spec, _w_spec],
    out_specs=_fullrow_spec,
    out_shape=jax.ShapeDtypeStruct((N, D), jnp.float32),
)


def kernel(x, edge_index, W1, Wl1, b1, W2, Wl2, b2):
    x_lo = x[:, :H]
    x_hi = x[:, H:]
    npadlen = EDGES_PAD - EDGES_PER_TILE
    pad_src = jnp.broadcast_to(
        jnp.arange(npadlen, dtype=jnp.int32) * 61 % N, (NTILES, npadlen))
    pad_dst = jnp.broadcast_to(
        N + (jnp.arange(npadlen, dtype=jnp.int32) % (NPAD - N)),
        (NTILES, npadlen))
    src_h = jnp.concatenate(
        [edge_index[0].reshape(NTILES, EDGES_PER_TILE), pad_src], axis=1
    ).reshape(NTILES, NCHUNKS, CHUNK)
    dst_h = jnp.concatenate(
        [edge_index[1].reshape(NTILES, EDGES_PER_TILE), pad_dst],
        axis=1,
    ).reshape(NTILES, NCHUNKS, CHUNK)
    idx_h = src_h.reshape(NTILES, NPAIRS, 2, CHUNK)
    zeros = jnp.zeros((ROWS_PER_TILE, H), jnp.float32)
    dummy = jnp.zeros((CHUNK, H), jnp.float32)
    b1r = b1.reshape(1, D)
    b2r = b2.reshape(1, D)

    t1 = _mm_pre1(x, Wl1, b1r)
    a1lo, a1hi = _sc_agg(x_lo, x_hi, idx_h, dst_h, zeros, dummy)
    h1lo, h1hi, t2 = _mm_mid(a1lo, a1hi, t1, W1, x, Wl2, b2r)
    a2lo, a2hi = _sc_agg(h1lo, h1hi, idx_h, dst_h, zeros, dummy)
    return _mm_post2(a2lo, a2hi, t2, W2)
